# Initial kernel scaffold; baseline (speedup 1.0000x reference)
import jax
import jax.numpy as jnp
from jax.experimental import pallas as pl


def kernel(features, labels, centers):
    raise NotImplementedError("write your pallas kernel here")


# speedup vs baseline: 1.2032x; 1.2032x over previous
import functools

import jax
import jax.numpy as jnp
from jax import lax
from jax.experimental import pallas as pl
from jax.experimental.pallas import tpu as pltpu
from jax.experimental.pallas import tpu_sc as plsc

_LANES = 16
_NC = 2
_NS = 16
_NW = _NC * _NS
_BATCH = 16384
_D = 128
_ROWS_PER_W = _BATCH // _NW
_CHUNK = 128
_NCHUNK = _ROWS_PER_W // _CHUNK
_VECS = _D // _LANES


def _sc_center_loss_partials(features, labels_2d, centers):
    mesh = plsc.VectorSubcoreMesh(core_axis_name="c", subcore_axis_name="s")

    @functools.partial(
        pl.kernel,
        out_type=jax.ShapeDtypeStruct((_NW, _LANES), jnp.float32),
        mesh=mesh,
        scratch_types=[
            pltpu.VMEM((_NCHUNK, _CHUNK), jnp.int32),
            pltpu.VMEM((2, _CHUNK, _D), jnp.float32),
            pltpu.VMEM((2, _CHUNK, _D), jnp.float32),
            pltpu.VMEM((_LANES,), jnp.float32),
            pltpu.SemaphoreType.DMA,
            pltpu.SemaphoreType.DMA,
            pltpu.SemaphoreType.DMA,
            pltpu.SemaphoreType.DMA,
        ],
    )
    def run(feat_hbm, idx_hbm, cent_hbm, out_hbm,
            idx_v, cent_v, feat_v, acc_v, sg0, sg1, sf0, sf1):
        wid = lax.axis_index("s") * _NC + lax.axis_index("c")
        base = wid * _ROWS_PER_W
        pltpu.sync_copy(idx_hbm.at[wid], idx_v)
        sgs = (sg0, sg1)
        sfs = (sf0, sf1)

        def start(c, b):
            g = pltpu.async_copy(cent_hbm.at[idx_v.at[c]], cent_v.at[b], sgs[b])
            f = pltpu.async_copy(
                feat_hbm.at[pl.ds(base + c * _CHUNK, _CHUNK)], feat_v.at[b],
                sfs[b])
            return g, f

        pending = start(0, 0)
        acc = tuple(jnp.zeros((_LANES,), jnp.float32) for _ in range(_VECS))
        for c in range(_NCHUNK):
            b = c % 2
            pending[0].wait()
            pending[1].wait()
            if c + 1 < _NCHUNK:
                pending = start(c + 1, (c + 1) % 2)

            def body(r, a, b=b):
                out = []
                for v in range(_VECS):
                    fv = feat_v[b, r, pl.ds(v * _LANES, _LANES)]
                    cv = cent_v[b, r, pl.ds(v * _LANES, _LANES)]
                    d = fv - cv
                    out.append(a[v] + d * d)
                return tuple(out)

            acc = lax.fori_loop(0, _CHUNK, body, acc)

        tot = acc[0]
        for v in range(1, _VECS):
            tot = tot + acc[v]
        acc_v[...] = tot
        pltpu.sync_copy(acc_v, out_hbm.at[wid])

    return run(features, labels_2d, centers)


def kernel(features, labels, centers):
    labels_2d = labels.astype(jnp.int32).reshape(_NW, _NCHUNK, _CHUNK)
    partials = _sc_center_loss_partials(features, labels_2d, centers)
    return jnp.sum(partials) * jnp.float32(0.5 / (2.0 * _BATCH))


# speedup vs baseline: 1.2091x; 1.0049x over previous
import functools

import jax
import jax.numpy as jnp
from jax import lax
from jax.experimental import pallas as pl
from jax.experimental.pallas import tpu as pltpu
from jax.experimental.pallas import tpu_sc as plsc

_LANES = 16
_NC = 2
_NS = 16
_NW = _NC * _NS
_BATCH = 16384
_D = 128
_ROWS_PER_W = _BATCH // _NW
_CHUNK = 128
_NCHUNK = _ROWS_PER_W // _CHUNK
_VECS = _D // _LANES


def _sc_center_loss_partials(features, labels_2d, centers):
    mesh = plsc.VectorSubcoreMesh(core_axis_name="c", subcore_axis_name="s")

    @functools.partial(
        pl.kernel,
        out_type=jax.ShapeDtypeStruct((_NW, _LANES), jnp.float32),
        mesh=mesh,
        scratch_types=[
            pltpu.VMEM((_NCHUNK, _CHUNK), jnp.int32),
            pltpu.VMEM((2, _CHUNK, _D), jnp.float32),
            pltpu.VMEM((2, _CHUNK, _D), jnp.float32),
            pltpu.VMEM((_LANES,), jnp.float32),
            pltpu.SemaphoreType.DMA,
            pltpu.SemaphoreType.DMA,
            pltpu.SemaphoreType.DMA,
            pltpu.SemaphoreType.DMA,
        ],
    )
    def run(feat_hbm, idx_hbm, cent_hbm, out_hbm,
            idx_v, cent_v, feat_v, acc_v, sg0, sg1, sf0, sf1):
        wid = lax.axis_index("s") * _NC + lax.axis_index("c")
        base = wid * _ROWS_PER_W
        pltpu.sync_copy(idx_hbm.at[wid], idx_v)
        sgs = (sg0, sg1)
        sfs = (sf0, sf1)

        def start(c, b):
            g = pltpu.async_copy(cent_hbm.at[idx_v.at[c]], cent_v.at[b], sgs[b])
            f = pltpu.async_copy(
                feat_hbm.at[pl.ds(base + c * _CHUNK, _CHUNK)], feat_v.at[b],
                sfs[b])
            return g, f

        pending = start(0, 0)
        acc = tuple(jnp.zeros((_LANES,), jnp.float32) for _ in range(_VECS))
        for c in range(_NCHUNK):
            b = c % 2
            pending[0].wait()
            pending[1].wait()
            if c + 1 < _NCHUNK:
                pending = start(c + 1, (c + 1) % 2)

            def body(r, a, b=b):
                out = []
                for v in range(_VECS):
                    fv = feat_v[b, r, pl.ds(v * _LANES, _LANES)]
                    cv = cent_v[b, r, pl.ds(v * _LANES, _LANES)]
                    d = fv - cv
                    out.append(a[v] + d * d)
                return tuple(out)

            acc = plsc.parallel_loop(0, _CHUNK, unroll=4, carry=acc)(body)

        tot = acc[0]
        for v in range(1, _VECS):
            tot = tot + acc[v]
        acc_v[...] = tot
        pltpu.sync_copy(acc_v, out_hbm.at[wid])

    return run(features, labels_2d, centers)


def kernel(features, labels, centers):
    labels_2d = labels.astype(jnp.int32).reshape(_NW, _NCHUNK, _CHUNK)
    partials = _sc_center_loss_partials(features, labels_2d, centers)
    return jnp.sum(partials) * jnp.float32(0.5 / (2.0 * _BATCH))


# speedup vs baseline: 1.2506x; 1.0343x over previous
import functools

import jax
import jax.numpy as jnp
from jax import lax
from jax.experimental import pallas as pl
from jax.experimental.pallas import tpu as pltpu
from jax.experimental.pallas import tpu_sc as plsc

_LANES = 16
_NC = 2
_NS = 16
_NW = _NC * _NS
_BATCH = 16384
_D = 128
_ROWS_PER_W = _BATCH // _NW
_CHUNKS = (128, 128, 128, 96, 32)
_MAXCHUNK = 128
_NBUF = 3
_VECS = _D // _LANES
_SCALE = 0.5 / (2.0 * _BATCH)


def _sc_center_loss_partials(features, labels, centers):
    mesh = plsc.VectorSubcoreMesh(core_axis_name="c", subcore_axis_name="s")

    @functools.partial(
        pl.kernel,
        out_type=jax.ShapeDtypeStruct((_NW, _LANES), jnp.float32),
        mesh=mesh,
        scratch_types=[
            pltpu.VMEM((_ROWS_PER_W,), jnp.int32),
            pltpu.VMEM((_NBUF, _MAXCHUNK, _D), jnp.float32),
            pltpu.VMEM((_NBUF, _MAXCHUNK, _D), jnp.float32),
            pltpu.VMEM((_LANES,), jnp.float32),
        ] + [pltpu.SemaphoreType.DMA] * (2 * _NBUF),
    )
    def run(feat_hbm, idx_hbm, cent_hbm, out_hbm,
            idx_v, cent_v, feat_v, acc_v, *sems):
        wid = lax.axis_index("s") * _NC + lax.axis_index("c")
        base = wid * _ROWS_PER_W
        pltpu.sync_copy(idx_hbm.at[pl.ds(base, _ROWS_PER_W)], idx_v)
        sg = sems[:_NBUF]
        sf = sems[_NBUF:]
        offs = []
        o = 0
        for n in _CHUNKS:
            offs.append(o)
            o += n

        def start(c):
            b = c % _NBUF
            n = _CHUNKS[c]
            o = offs[c]
            g = pltpu.async_copy(
                cent_hbm.at[idx_v.at[pl.ds(o, n)]],
                cent_v.at[b, pl.ds(0, n)], sg[b])
            f = pltpu.async_copy(
                feat_hbm.at[pl.ds(base + o, n)],
                feat_v.at[b, pl.ds(0, n)], sf[b])
            return g, f

        pending = [start(c) for c in range(_NBUF)]
        acc = tuple(jnp.zeros((_LANES,), jnp.float32) for _ in range(_VECS))
        for c in range(len(_CHUNKS)):
            b = c % _NBUF
            g, f = pending[b]
            g.wait()
            f.wait()

            def body(r, a, b=b):
                out = []
                for v in range(_VECS):
                    fv = feat_v[b, r, pl.ds(v * _LANES, _LANES)]
                    cv = cent_v[b, r, pl.ds(v * _LANES, _LANES)]
                    d = fv - cv
                    out.append(a[v] + d * d)
                return tuple(out)

            acc = plsc.parallel_loop(0, _CHUNKS[c], unroll=4, carry=acc)(body)
            if c + _NBUF < len(_CHUNKS):
                pending[b] = start(c + _NBUF)

        tot = acc[0]
        for v in range(1, _VECS):
            tot = tot + acc[v]
        acc_v[...] = tot * jnp.float32(_SCALE)
        pltpu.sync_copy(acc_v, out_hbm.at[wid])

    return run(features, labels, centers)


def kernel(features, labels, centers):
    labels_i32 = labels.astype(jnp.int32)
    partials = _sc_center_loss_partials(features, labels_i32, centers)
    return jnp.sum(partials)


# speedup vs baseline: 1.2906x; 1.0320x over previous
import functools

import jax
import jax.numpy as jnp
from jax import lax
from jax.experimental import pallas as pl
from jax.experimental.pallas import tpu as pltpu
from jax.experimental.pallas import tpu_sc as plsc

_LANES = 16
_NC = 2
_NS = 16
_NW = _NC * _NS
_BATCH = 16384
_D = 128
_ROWS_PER_W = _BATCH // _NW
_CHUNKS = (32, 96, 128, 128, 96, 32)
_MAXCHUNK = 128
_NBUF = 3
_VECS = _D // _LANES
_SCALE = 0.5 / (2.0 * _BATCH)


def _sc_center_loss_partials(features, labels, centers):
    mesh = plsc.VectorSubcoreMesh(core_axis_name="c", subcore_axis_name="s")

    @functools.partial(
        pl.kernel,
        out_type=jax.ShapeDtypeStruct((_NW, _LANES), jnp.float32),
        mesh=mesh,
        scratch_types=[
            pltpu.VMEM((_ROWS_PER_W,), jnp.int32),
            pltpu.VMEM((_NBUF, _MAXCHUNK, _D), jnp.float32),
            pltpu.VMEM((_NBUF, _MAXCHUNK, _D), jnp.float32),
            pltpu.VMEM((_LANES,), jnp.float32),
        ] + [pltpu.SemaphoreType.DMA] * (2 * _NBUF),
    )
    def run(feat_hbm, idx_hbm, cent_hbm, out_hbm,
            idx_v, cent_v, feat_v, acc_v, *sems):
        wid = lax.axis_index("s") * _NC + lax.axis_index("c")
        base = wid * _ROWS_PER_W
        pltpu.sync_copy(idx_hbm.at[pl.ds(base, _ROWS_PER_W)], idx_v)
        sg = sems[:_NBUF]
        sf = sems[_NBUF:]
        offs = []
        o = 0
        for n in _CHUNKS:
            offs.append(o)
            o += n

        def start(c):
            b = c % _NBUF
            n = _CHUNKS[c]
            o = offs[c]
            g = pltpu.async_copy(
                cent_hbm.at[idx_v.at[pl.ds(o, n)]],
                cent_v.at[b, pl.ds(0, n)], sg[b])
            f = pltpu.async_copy(
                feat_hbm.at[pl.ds(base + o, n)],
                feat_v.at[b, pl.ds(0, n)], sf[b])
            return g, f

        pending = [start(c) for c in range(_NBUF)]
        acc = tuple(jnp.zeros((_LANES,), jnp.float32) for _ in range(_VECS))
        for c in range(len(_CHUNKS)):
            b = c % _NBUF
            g, f = pending[b]
            g.wait()
            f.wait()

            def body(r, a, b=b):
                out = []
                for v in range(_VECS):
                    fv = feat_v[b, r, pl.ds(v * _LANES, _LANES)]
                    cv = cent_v[b, r, pl.ds(v * _LANES, _LANES)]
                    d = fv - cv
                    out.append(a[v] + d * d)
                return tuple(out)

            acc = plsc.parallel_loop(0, _CHUNKS[c], unroll=4, carry=acc)(body)
            if c + _NBUF < len(_CHUNKS):
                pending[b] = start(c + _NBUF)

        tot = acc[0]
        for v in range(1, _VECS):
            tot = tot + acc[v]
        acc_v[...] = tot * jnp.float32(_SCALE)
        pltpu.sync_copy(acc_v, out_hbm.at[wid])

    return run(features, labels, centers)


def kernel(features, labels, centers):
    labels_i32 = labels.astype(jnp.int32)
    partials = _sc_center_loss_partials(features, labels_i32, centers)
    return jnp.sum(partials)
